# Initial kernel scaffold; baseline (speedup 1.0000x reference)
import jax
import jax.numpy as jnp
from jax.experimental import pallas as pl


def kernel(edge_index, features, preference, W1, b1, W2, b2):
    raise NotImplementedError("write your pallas kernel here")


# speedup vs baseline: 18.9334x; 18.9334x over previous
import functools

import jax
import jax.numpy as jnp
from jax import lax
from jax.experimental import pallas as pl
from jax.experimental.pallas import tpu as pltpu
from jax.experimental.pallas import tpu_sc as plsc

NUM_USER = 10000
NUM_ITEM = 40000
N_NODES = NUM_USER + NUM_ITEM
N_EDGES = 800000
DIM_FEAT = 256
DIM_LATENT = 64
HALF = DIM_LATENT // 2

NC = 2
NS = 16
NPAD = 51200
SLICE = NPAD // NS
EPAD = 819200
ECHUNKS = EPAD // 128

P_CH_PER_SUB = ECHUNKS // NS
P_SB = 40
P_NSLOT = 4
P_OUTER = P_CH_PER_SUB // P_SB

D_CH_PER_W = ECHUNKS // (NC * NS)
D_K = 8
D_OUTER = D_CH_PER_W // D_K

@functools.cache
def _mesh():
    return plsc.VectorSubcoreMesh(
        core_axis_name="c", subcore_axis_name="s", num_cores=NC, num_subcores=NS
    )


def _deg_body(ei_ref, zeros1_ref, degp_ref, dacc, idx, ones_v, sem):
    c = lax.axis_index("c")
    s = lax.axis_index("s")
    w = c * NS + s
    r0 = s * SLICE
    for t in range(8):
        ones_v[pl.ds(t * 16, 16)] = jnp.ones((16,), jnp.float32)
    pltpu.sync_copy(zeros1_ref.at[pl.ds(r0, SLICE)], dacc.at[pl.ds(r0, SLICE)])
    plsc.subcore_barrier()

    def outer(o, carry):
        cb = w * D_CH_PER_W + o * D_K
        pltpu.sync_copy(ei_ref.at[0].at[pl.ds(cb, D_K)], idx)
        descs = [
            pltpu.async_copy(ones_v, dacc.at[idx.at[j]], sem, add=True)
            for j in range(D_K)
        ]
        for d in descs:
            d.wait()
        return carry

    lax.fori_loop(0, D_OUTER, outer, 0)
    plsc.subcore_barrier()
    pltpu.sync_copy(dacc.at[pl.ds(r0, SLICE)], degp_ref.at[c].at[pl.ds(r0, SLICE)])


@functools.cache
def _deg_kernel():
    return pl.kernel(
        _deg_body,
        out_type=jax.ShapeDtypeStruct((NC, NPAD), jnp.float32),
        mesh=_mesh(),
        scratch_types=[
            pltpu.VMEM_SHARED((NPAD,), jnp.float32),
            pltpu.VMEM((D_K, 128), jnp.int32),
            pltpu.VMEM((128,), jnp.float32),
            pltpu.SemaphoreType.DMA,
        ],
    )


def _deg_call(ei, zeros1):
    return _deg_kernel()(ei, zeros1)


def _edge_loop(tab, ei_ref, s, acc, idxr, idxc, bufs, semgs, semss):
    def fire_gather(t):
        sl = t % P_NSLOT
        return pltpu.async_copy(tab.at[idxr.at[t]], bufs.at[sl], semgs.at[sl])

    def fire_scatter(t):
        sl = t % P_NSLOT
        return pltpu.async_copy(bufs.at[sl], acc.at[idxc.at[t]], semss.at[sl],
                                add=True)

    def outer(o, carry):
        cb = s * P_CH_PER_SUB + o * P_SB
        pltpu.sync_copy(ei_ref.at[0].at[pl.ds(cb, P_SB)], idxr)
        pltpu.sync_copy(ei_ref.at[1].at[pl.ds(cb, P_SB)], idxc)
        g_pend = {t: fire_gather(t) for t in range(P_NSLOT - 1)}
        s_pend = {}
        for t in range(P_SB):
            g_pend.pop(t).wait()
            s_pend[t] = fire_scatter(t)
            nxt = t + P_NSLOT - 1
            if nxt < P_SB:
                if t >= 1:
                    s_pend.pop(t - 1).wait()
                g_pend[nxt] = fire_gather(nxt)
        for t in sorted(s_pend):
            s_pend.pop(t).wait()
        return carry

    lax.fori_loop(0, P_OUTER, outer, 0)


def _gcn2_body(tab_ref, ei_ref, d2_ref, xpp_ref, s12_ref,
               acc, idxr, idxc, bufs, d2v, semgs, semss):
    c = lax.axis_index("c")
    s = lax.axis_index("s")
    r0 = s * SLICE
    z = jnp.zeros((16,), jnp.float32)
    for r in range(128):
        bufs[0, r, pl.ds(0, 16)] = z
        bufs[0, r, pl.ds(16, 16)] = z

    def zchunk(k, carry):
        pltpu.sync_copy(bufs.at[0], acc.at[pl.ds(r0 + k * 128, 128)])
        return carry

    lax.fori_loop(0, SLICE // 128, zchunk, 0)
    plsc.subcore_barrier()

    _edge_loop(tab_ref.at[c], ei_ref, s, acc, idxr, idxc, bufs, semgs, semss)
    plsc.subcore_barrier()

    def sc_chunk(k, carry):
        row = r0 + k * 128
        pltpu.sync_copy(acc.at[pl.ds(row, 128)], bufs.at[0])
        pltpu.sync_copy(d2_ref.at[pl.ds(s * (SLICE // 128) + k, 1)], d2v)
        for g in range(8):
            dvec = d2v[0, pl.ds(g * 16, 16)]
            for r16 in range(16):
                r = g * 16 + r16
                dscal = dvec[r16]
                bufs[0, r, pl.ds(0, 16)] = bufs[0, r, pl.ds(0, 16)] * dscal
                bufs[0, r, pl.ds(16, 16)] = bufs[0, r, pl.ds(16, 16)] * dscal
        pltpu.sync_copy(bufs.at[0], xpp_ref.at[c].at[pl.ds(row, 128)])
        return carry

    lax.fori_loop(0, SLICE // 128, sc_chunk, 0)
    plsc.subcore_barrier()

    _edge_loop(xpp_ref.at[c], ei_ref, s, acc, idxr, idxc, bufs, semgs, semss)
    plsc.subcore_barrier()
    pltpu.sync_copy(acc.at[pl.ds(r0, SLICE)], s12_ref.at[c].at[pl.ds(r0, SLICE)])


@functools.cache
def _gcn2_kernel():
    return pl.kernel(
        _gcn2_body,
        out_type=(
            jax.ShapeDtypeStruct((NC, NPAD, HALF), jnp.float32),
            jax.ShapeDtypeStruct((NC, NPAD, HALF), jnp.float32),
        ),
        mesh=_mesh(),
        compiler_params=pltpu.CompilerParams(use_tc_tiling_on_sc=False),
        scratch_types=[
            pltpu.VMEM_SHARED((NPAD, HALF), jnp.float32),
            pltpu.VMEM((P_SB, 128), jnp.int32),
            pltpu.VMEM((P_SB, 128), jnp.int32),
            pltpu.VMEM((P_NSLOT, 128, HALF), jnp.float32),
            pltpu.VMEM((1, 128), jnp.float32),
            pltpu.SemaphoreType.DMA((P_NSLOT,)),
            pltpu.SemaphoreType.DMA((P_NSLOT,)),
        ],
    )


def _gcn2_call(tab, ei, d2):
    return _gcn2_kernel()(tab, ei, d2)


_B = 5000
_NBLK = N_NODES // _B
_PBLK = NUM_USER // _B


def _mlp_body(pref_ref, feat_ref, w1_ref, b1_ref, w2_ref, b2_ref,
              x_ref, tmp_ref):
    i = pl.program_id(0)

    @pl.when(i < _PBLK)
    def _():
        tmp_ref[...] = pref_ref[...]

    @pl.when(i >= _PBLK)
    def _():
        f = feat_ref[...]
        z = jnp.dot(f, w1_ref[...], preferred_element_type=jnp.float32)
        z = z + b1_ref[...]
        z = jnp.where(z >= 0, z, 0.01 * z)
        t = jnp.dot(z, w2_ref[...], preferred_element_type=jnp.float32)
        tmp_ref[...] = t + b2_ref[...]

    t = tmp_ref[...]
    n = jnp.sqrt(jnp.sum(t * t, axis=1, keepdims=True))
    x_ref[...] = t / jnp.maximum(n, 1e-12)


def _mlp(pref, feat, w1, b1, w2, b2):
    return pl.pallas_call(
        _mlp_body,
        grid=(_NBLK,),
        in_specs=[
            pl.BlockSpec((_B, DIM_LATENT), lambda i: (jnp.minimum(i, _PBLK - 1), 0)),
            pl.BlockSpec((_B, DIM_FEAT), lambda i: (jnp.maximum(i - _PBLK, 0), 0)),
            pl.BlockSpec((DIM_FEAT, DIM_FEAT), lambda i: (0, 0)),
            pl.BlockSpec((1, DIM_FEAT), lambda i: (0, 0)),
            pl.BlockSpec((DIM_FEAT, DIM_LATENT), lambda i: (0, 0)),
            pl.BlockSpec((1, DIM_LATENT), lambda i: (0, 0)),
        ],
        out_specs=pl.BlockSpec((_B, DIM_LATENT), lambda i: (i, 0)),
        out_shape=jax.ShapeDtypeStruct((N_NODES, DIM_LATENT), jnp.float32),
        scratch_shapes=[pltpu.VMEM((_B, DIM_LATENT), jnp.float32)],
    )(pref, feat, w1, b1, w2, b2)


def _prep2_body(x_ref, degp_ref, dinve_ref, xps_ref):
    i = pl.program_id(0)
    degl = degp_ref[:, 0, :] + degp_ref[:, 1, :]
    corr = jnp.where(
        (jax.lax.broadcasted_iota(jnp.int32, (1, _B), 1) == 0) & (i == 0),
        float(EPAD - N_EDGES), 0.0)
    degl = degl - corr
    dinvl = jnp.where(degl > 0, lax.rsqrt(degl), 0.0)
    dinve = jnp.broadcast_to(dinvl, (DIM_LATENT, _B)).T
    dinve_ref[...] = dinve
    xp = dinve * x_ref[...]
    xps_ref[0] = xp[:, :HALF]
    xps_ref[1] = xp[:, HALF:]


def _prep2(x, degp):
    return pl.pallas_call(
        _prep2_body,
        grid=(_NBLK,),
        in_specs=[
            pl.BlockSpec((_B, DIM_LATENT), lambda i: (i, 0)),
            pl.BlockSpec((1, NC, _B), lambda i: (i, 0, 0)),
        ],
        out_specs=[
            pl.BlockSpec((_B, DIM_LATENT), lambda i: (i, 0)),
            pl.BlockSpec((NC, _B, HALF), lambda i: (0, i, 0)),
        ],
        out_shape=[
            jax.ShapeDtypeStruct((N_NODES, DIM_LATENT), jnp.float32),
            jax.ShapeDtypeStruct((NC, NPAD, HALF), jnp.float32),
        ],
    )(x, degp)


def _d2_body(degp_ref, d2_ref):
    i = pl.program_id(0)
    deg = degp_ref[0] + degp_ref[1]
    corr = jnp.where(
        (jax.lax.broadcasted_iota(jnp.int32, (8, 128), 0) == 0)
        & (jax.lax.broadcasted_iota(jnp.int32, (8, 128), 1) == 0) & (i == 0),
        float(EPAD - N_EDGES), 0.0)
    deg = deg - corr
    d2_ref[...] = jnp.where(deg > 0, 1.0 / deg, 0.0)


def _d2(degp2):
    return pl.pallas_call(
        _d2_body,
        grid=(NPAD // 1024,),
        in_specs=[pl.BlockSpec((NC, 8, 128), lambda i: (0, i, 0))],
        out_specs=pl.BlockSpec((8, 128), lambda i: (i, 0)),
        out_shape=jax.ShapeDtypeStruct((NPAD // 128, 128), jnp.float32),
    )(degp2)


def _final_body(x_ref, s12_ref, dinve_ref, out_ref):
    x = x_ref[...]
    d = dinve_ref[...]
    left = x[:, :HALF] + d[:, :HALF] * s12_ref[0]
    right = x[:, HALF:] + d[:, HALF:] * s12_ref[1]
    out_ref[...] = jnp.concatenate([left, right], axis=1)


def _final(x, s12, dinve):
    return pl.pallas_call(
        _final_body,
        grid=(_NBLK,),
        in_specs=[
            pl.BlockSpec((_B, DIM_LATENT), lambda i: (i, 0)),
            pl.BlockSpec((NC, _B, HALF), lambda i: (0, i, 0)),
            pl.BlockSpec((_B, DIM_LATENT), lambda i: (i, 0)),
        ],
        out_specs=pl.BlockSpec((_B, DIM_LATENT), lambda i: (i, 0)),
        out_shape=jax.ShapeDtypeStruct((N_NODES, DIM_LATENT), jnp.float32),
    )(x, s12, dinve)


def kernel(edge_index, features, preference, W1, b1, W2, b2):
    npadE = EPAD - N_EDGES
    trash = N_NODES + (jnp.arange(npadE, dtype=jnp.int32) % (NPAD - N_NODES))
    pad2 = jnp.stack([jnp.zeros((npadE,), jnp.int32), trash])
    ei = jnp.concatenate([edge_index, pad2], axis=1).reshape(2, ECHUNKS, 128)

    zeros1 = jnp.zeros((NPAD,), jnp.float32)

    degp = _deg_call(ei, zeros1)
    degp3 = degp[:, :N_NODES].reshape(NC, _NBLK, _B).transpose(1, 0, 2)
    d2 = _d2(degp.reshape(NC, NPAD // 128, 128))

    x = _mlp(preference, features, W1, b1.reshape(1, DIM_FEAT),
             W2, b2.reshape(1, DIM_LATENT))
    dinve, xps = _prep2(x, degp3)

    _, s12 = _gcn2_call(xps, ei, d2)
    x_hat = _final(x, s12, dinve)
    return (x_hat, preference)
